# baseline (speedup 1.0000x reference)
import jax
import jax.numpy as jnp
from jax import lax
from jax.experimental import pallas as pl
from jax.experimental.pallas import tpu as pltpu
from jax.experimental.pallas import tpu_sc as plsc

_B = 16384
_D = 64
_V = 100000
_NW = 32
_NC = 2
_WIN = 768
_STRIDE = 768
_LAST = 99328
_NWIN = 131
_L = 16
_BAT = 128
_SROWS = _B + _NW * _L


def _phase_a(ut_hbm, vt_hbm, uid_hbm, iid_hbm, ug_hbm, vg_hbm,
             uids_v, iids_v, list_u, list_v, sub_v, blk_v, sb_v, bidx_v,
             sem, sem2):
    wid = lax.axis_index("s") * _NC + lax.axis_index("c")
    lanes = lax.iota(jnp.int32, _L)
    k0 = wid * _NWIN // _NW
    k1 = (wid + 1) * _NWIN // _NW
    ro = k0 * _STRIDE
    rend = jnp.minimum((k1 - 1) * _STRIDE, _LAST) + _WIN
    sentinel = (_B + wid * _L + lanes) * 4096 + 4095

    def outer(mc, nuv):
        pltpu.sync_copy(uid_hbm.at[pl.ds(mc * 1024, 1024)], uids_v)
        pltpu.sync_copy(iid_hbm.at[pl.ds(mc * 1024, 1024)], iids_v)
        base = mc * 1024

        def inner(g2, nuv):
            nu, nv = nuv
            for h in range(2):
                bvec = base + (g2 * 2 + h) * _L + lanes
                uvec = uids_v[pl.ds((g2 * 2 + h) * _L, _L)]
                um = (uvec >= ro) & (uvec < rend)
                plsc.store_compressed(list_u.at[pl.ds(nu, _L)],
                                      bvec * 4096 + (uvec - ro), mask=um)
                ivec = iids_v[pl.ds((g2 * 2 + h) * _L, _L)]
                im = (ivec >= ro) & (ivec < rend)
                plsc.store_compressed(list_v.at[pl.ds(nv, _L)],
                                      bvec * 4096 + (ivec - ro), mask=im)
                nu = nu + plsc.all_reduce_population_count(um)[0]
                nv = nv + plsc.all_reduce_population_count(im)[0]
            return (nu, nv)

        return lax.fori_loop(0, 1024 // (2 * _L), inner, nuv)

    with jax.named_scope("ph_scan"):
        nu, nv = lax.fori_loop(0, _B // 1024, outer,
                               (jnp.int32(0), jnp.int32(0)))
    list_u[pl.ds(nu, _L)] = sentinel
    list_v[pl.ds(nv, _L)] = sentinel

    def do_table(src_hbm, lst, n, out_hbm):
        def window(k, carry):
            ok = jnp.minimum(k * _STRIDE, _LAST)
            ok = pl.multiple_of(ok, 128)
            with jax.named_scope("ph_stream"):
                pltpu.async_copy(
                    src_hbm.at[:, pl.ds(ok, _WIN)], blk_v, sem).wait()
            od = ok - ro

            def subcomp(g, ns):
                p = lst[pl.ds(g * _L, _L)]
                c = lax.rem(p, 4096)
                mask = (c >= od) & (c < od + _WIN)
                plsc.store_compressed(sub_v.at[pl.ds(ns, _L)], p, mask=mask)
                return ns + plsc.all_reduce_population_count(mask)[0]

            with jax.named_scope("ph_subcomp"):
                ns = lax.fori_loop(0, (n + _L - 1) // _L, subcomp,
                                   jnp.int32(0))
            for t in range(_BAT // _L):
                sub_v[pl.ds(ns + t * _L, _L)] = sentinel

            def batch(bt, carry2):
                @pl.when(bt > 0)
                def _():
                    pltpu.make_async_copy(
                        ug_hbm.at[pl.ds(0, _BAT)], sb_v, sem2).wait()
                for gi in range(_BAT // _L):
                    p = sub_v[pl.ds(bt * _BAT + gi * _L, _L)]
                    b = lax.shift_right_logical(p, 12)
                    cloc = lax.rem(p, 4096) - od
                    cloc = jnp.minimum(jnp.maximum(cloc, jnp.int32(0)),
                                       jnp.int32(_WIN - 1))
                    rows = gi * _L + lanes
                    for f in range(_D):
                        g = plsc.load_gather(
                            blk_v, [jnp.full((_L,), f, jnp.int32), cloc])
                        plsc.store_scatter(
                            sb_v, [rows, lax.bitwise_and(b + f, 63)], g)
                    bidx_v[pl.ds(gi * _L, _L)] = b
                pltpu.async_copy(sb_v, out_hbm.at[bidx_v], sem2)
                return carry2

            nbat = (ns + _BAT - 1) // _BAT
            with jax.named_scope("ph_extract"):
                lax.fori_loop(0, nbat, batch, 0)
            @pl.when(nbat > 0)
            def _():
                pltpu.make_async_copy(
                    ug_hbm.at[pl.ds(0, _BAT)], sb_v, sem2).wait()
            return carry

        lax.fori_loop(k0, k1, window, 0)

    do_table(ut_hbm, list_u, nu, ug_hbm)
    do_table(vt_hbm, list_v, nv, vg_hbm)


def _phase_b(ug_ref, vg_ref, out_ref):
    u = ug_ref[:, : _D]
    v = vg_ref[:, : _D]
    out_ref[:] = jnp.sum(u * v, axis=1)


def kernel(user_table, item_table, user_ids, item_ids):
    mesh = plsc.VectorSubcoreMesh(core_axis_name="c", subcore_axis_name="s")
    ka = pl.kernel(
        _phase_a,
        mesh=mesh,
        out_type=(
            jax.ShapeDtypeStruct((_SROWS, 128), jnp.float32),
            jax.ShapeDtypeStruct((_SROWS, 128), jnp.float32),
        ),
        scratch_types=[
            pltpu.VMEM((1024,), jnp.int32),
            pltpu.VMEM((1024,), jnp.int32),
            pltpu.VMEM((_B + _L,), jnp.int32),
            pltpu.VMEM((_B + _L,), jnp.int32),
            pltpu.VMEM((_B + _BAT,), jnp.int32),
            pltpu.VMEM((_D, _WIN), jnp.float32),
            pltpu.VMEM((_BAT, 128), jnp.float32),
            pltpu.VMEM((_BAT,), jnp.int32),
            pltpu.SemaphoreType.DMA,
            pltpu.SemaphoreType.DMA,
        ],
        compiler_params=pltpu.CompilerParams(needs_layout_passes=False),
    )
    ug, vg = ka(user_table.T, item_table.T,
                user_ids.astype(jnp.int32), item_ids.astype(jnp.int32))

    kb = pl.pallas_call(
        _phase_b,
        grid=(_B // 512,),
        in_specs=[
            pl.BlockSpec((512, 128), lambda i: (i, 0)),
            pl.BlockSpec((512, 128), lambda i: (i, 0)),
        ],
        out_specs=pl.BlockSpec((512,), lambda i: (i,)),
        out_shape=jax.ShapeDtypeStruct((_B,), jnp.float32),
    )
    return kb(ug, vg)


# speedup vs baseline: 1.1558x; 1.1558x over previous
import jax
import jax.numpy as jnp
from jax import lax
from jax.experimental import pallas as pl
from jax.experimental.pallas import tpu as pltpu
from jax.experimental.pallas import tpu_sc as plsc

_B = 16384
_D = 64
_V = 100000
_NW = 32
_NC = 2
_WIN = 384
_STRIDE = 384
_LAST = 99712
_NWIN = 261
_L = 16
_BAT = 64
_SROWS = _B + _NW * _L


def _phase_a(ut_hbm, vt_hbm, uid_hbm, iid_hbm, ug_hbm, vg_hbm,
             uids_v, iids_v, list_u, list_v, sub_v, blk_a, blk_b, sb_v,
             bidx_v, sem_a, sem_b, sem2):
    wid = lax.axis_index("s") * _NC + lax.axis_index("c")
    lanes = lax.iota(jnp.int32, _L)
    k0 = wid * _NWIN // _NW
    k1 = (wid + 1) * _NWIN // _NW
    ro = k0 * _STRIDE
    rend = jnp.minimum((k1 - 1) * _STRIDE, _LAST) + _WIN
    sentinel = (_B + wid * _L + lanes) * 4096 + 4095

    def outer(mc, nuv):
        pltpu.sync_copy(uid_hbm.at[pl.ds(mc * 1024, 1024)], uids_v)
        pltpu.sync_copy(iid_hbm.at[pl.ds(mc * 1024, 1024)], iids_v)
        base = mc * 1024

        def inner(g2, nuv):
            nu, nv = nuv
            for h in range(2):
                bvec = base + (g2 * 2 + h) * _L + lanes
                uvec = uids_v[pl.ds((g2 * 2 + h) * _L, _L)]
                um = (uvec >= ro) & (uvec < rend)
                plsc.store_compressed(list_u.at[pl.ds(nu, _L)],
                                      bvec * 4096 + (uvec - ro), mask=um)
                ivec = iids_v[pl.ds((g2 * 2 + h) * _L, _L)]
                im = (ivec >= ro) & (ivec < rend)
                plsc.store_compressed(list_v.at[pl.ds(nv, _L)],
                                      bvec * 4096 + (ivec - ro), mask=im)
                nu = nu + plsc.all_reduce_population_count(um)[0]
                nv = nv + plsc.all_reduce_population_count(im)[0]
            return (nu, nv)

        return lax.fori_loop(0, 1024 // (2 * _L), inner, nuv)

    nu, nv = lax.fori_loop(0, _B // 1024, outer,
                           (jnp.int32(0), jnp.int32(0)))
    list_u[pl.ds(nu, _L)] = sentinel
    list_v[pl.ds(nv, _L)] = sentinel

    def _start(ok):
        return pl.multiple_of(jnp.minimum(ok * _STRIDE, _LAST), 128)

    def do_table(src_hbm, lst, n, out_hbm):
        def fire(k, blk, semx):
            pltpu.async_copy(src_hbm.at[:, pl.ds(_start(k), _WIN)], blk, semx)

        def drain_blk(blk, semx):
            pltpu.make_async_copy(
                src_hbm.at[:, pl.ds(0, _WIN)], blk, semx).wait()

        def process(k, blk):
            ok = _start(k)
            od = ok - ro

            def subcomp(g, ns):
                p = lst[pl.ds(g * _L, _L)]
                c = lax.rem(p, 4096)
                mask = (c >= od) & (c < od + _WIN)
                plsc.store_compressed(sub_v.at[pl.ds(ns, _L)], p, mask=mask)
                return ns + plsc.all_reduce_population_count(mask)[0]

            ns = lax.fori_loop(0, (n + _L - 1) // _L, subcomp, jnp.int32(0))
            for t in range(_BAT // _L):
                sub_v[pl.ds(ns + t * _L, _L)] = sentinel

            def batch(bt, carry2):
                @pl.when(bt > 0)
                def _():
                    pltpu.make_async_copy(
                        ug_hbm.at[pl.ds(0, _BAT)], sb_v, sem2).wait()
                for gi in range(_BAT // _L):
                    p = sub_v[pl.ds(bt * _BAT + gi * _L, _L)]
                    b = lax.shift_right_logical(p, 12)
                    cloc = lax.rem(p, 4096) - od
                    cloc = jnp.minimum(jnp.maximum(cloc, jnp.int32(0)),
                                       jnp.int32(_WIN - 1))
                    rows = gi * _L + lanes
                    for f in range(_D):
                        g = plsc.load_gather(
                            blk, [jnp.full((_L,), f, jnp.int32), cloc])
                        plsc.store_scatter(
                            sb_v, [rows, lax.bitwise_and(b + f, 63)], g)
                    bidx_v[pl.ds(gi * _L, _L)] = b
                pltpu.async_copy(sb_v, out_hbm.at[bidx_v], sem2)
                return carry2

            nbat = (ns + _BAT - 1) // _BAT
            lax.fori_loop(0, nbat, batch, 0)
            @pl.when(nbat > 0)
            def _():
                pltpu.make_async_copy(
                    ug_hbm.at[pl.ds(0, _BAT)], sb_v, sem2).wait()

        fire(k0, blk_a, sem_a)

        def pair(p, carry):
            ka = k0 + 2 * p
            drain_blk(blk_a, sem_a)

            @pl.when(ka + 1 < k1)
            def _():
                fire(ka + 1, blk_b, sem_b)
            process(ka, blk_a)

            @pl.when(ka + 1 < k1)
            def _():
                drain_blk(blk_b, sem_b)

                @pl.when(ka + 2 < k1)
                def _():
                    fire(ka + 2, blk_a, sem_a)
                process(ka + 1, blk_b)
            return carry

        lax.fori_loop(0, (k1 - k0 + 1) // 2, pair, 0)

    do_table(ut_hbm, list_u, nu, ug_hbm)
    do_table(vt_hbm, list_v, nv, vg_hbm)


def _phase_b(ug_ref, vg_ref, out_ref):
    u = ug_ref[:, : _D]
    v = vg_ref[:, : _D]
    out_ref[:] = jnp.sum(u * v, axis=1)


def kernel(user_table, item_table, user_ids, item_ids):
    mesh = plsc.VectorSubcoreMesh(core_axis_name="c", subcore_axis_name="s")
    ka = pl.kernel(
        _phase_a,
        mesh=mesh,
        out_type=(
            jax.ShapeDtypeStruct((_SROWS, 128), jnp.float32),
            jax.ShapeDtypeStruct((_SROWS, 128), jnp.float32),
        ),
        scratch_types=[
            pltpu.VMEM((1024,), jnp.int32),
            pltpu.VMEM((1024,), jnp.int32),
            pltpu.VMEM((_B + _L,), jnp.int32),
            pltpu.VMEM((_B + _L,), jnp.int32),
            pltpu.VMEM((_B + _BAT,), jnp.int32),
            pltpu.VMEM((_D, _WIN), jnp.float32),
            pltpu.VMEM((_D, _WIN), jnp.float32),
            pltpu.VMEM((_BAT, 128), jnp.float32),
            pltpu.VMEM((_BAT,), jnp.int32),
            pltpu.SemaphoreType.DMA,
            pltpu.SemaphoreType.DMA,
            pltpu.SemaphoreType.DMA,
        ],
        compiler_params=pltpu.CompilerParams(needs_layout_passes=False),
    )
    ug, vg = ka(user_table.T, item_table.T,
                user_ids.astype(jnp.int32), item_ids.astype(jnp.int32))

    kb = pl.pallas_call(
        _phase_b,
        grid=(_B // 512,),
        in_specs=[
            pl.BlockSpec((512, 128), lambda i: (i, 0)),
            pl.BlockSpec((512, 128), lambda i: (i, 0)),
        ],
        out_specs=pl.BlockSpec((512,), lambda i: (i,)),
        out_shape=jax.ShapeDtypeStruct((_B,), jnp.float32),
    )
    return kb(ug, vg)


# speedup vs baseline: 1.2313x; 1.0653x over previous
import jax
import jax.numpy as jnp
from jax import lax
from jax.experimental import pallas as pl
from jax.experimental.pallas import tpu as pltpu
from jax.experimental.pallas import tpu_sc as plsc

_B = 16384
_D = 64
_NC = 2
_NS = 16
_NW = _NC * _NS
_BPW = _B // _NW
_CHUNK = 128
_NCHUNK = _BPW // _CHUNK
_L = 16
_GROUPS = _BPW // _L


def _mf_body(user_hbm, item_hbm, uid_hbm, iid_hbm, out_hbm,
             uidx_v, iidx_v, urows_v, irows_v, part_v, out_v, sem_u, sem_v):
    wid = lax.axis_index("s") * _NC + lax.axis_index("c")
    base = wid * _BPW

    for c in range(_NCHUNK):
        pltpu.sync_copy(uid_hbm.at[pl.ds(base + c * _CHUNK, _CHUNK)], uidx_v.at[c])
        pltpu.sync_copy(iid_hbm.at[pl.ds(base + c * _CHUNK, _CHUNK)], iidx_v.at[c])

    copies = []
    for c in range(_NCHUNK):
        copies.append(pltpu.async_copy(
            user_hbm.at[uidx_v.at[c]], urows_v.at[pl.ds(c * _CHUNK, _CHUNK)], sem_u))
        copies.append(pltpu.async_copy(
            item_hbm.at[iidx_v.at[c]], irows_v.at[pl.ds(c * _CHUNK, _CHUNK)], sem_v))
    for cp in copies:
        cp.wait()

    lanes = lax.iota(jnp.int32, 16)

    def group_body(g, carry):
        row0 = g * _L
        for r in range(_L):
            row = row0 + r
            acc = urows_v[row, pl.ds(0, 16)] * irows_v[row, pl.ds(0, 16)]
            for c in range(1, _D // 16):
                acc = acc + (urows_v[row, pl.ds(c * 16, 16)]
                             * irows_v[row, pl.ds(c * 16, 16)])
            part_v[r] = acc
        s = jnp.zeros((16,), jnp.float32)
        for col in range(_L):
            s = s + plsc.load_gather(
                part_v, [lanes, jnp.full((16,), col, jnp.int32)])
        out_v[pl.ds(row0, _L)] = s
        return carry

    lax.fori_loop(0, _GROUPS, group_body, 0)
    pltpu.sync_copy(out_v, out_hbm.at[pl.ds(base, _BPW)])


def kernel(user_table, item_table, user_ids, item_ids):
    mesh = plsc.VectorSubcoreMesh(core_axis_name="c", subcore_axis_name="s")
    k = pl.kernel(
        _mf_body,
        mesh=mesh,
        out_type=jax.ShapeDtypeStruct((_B,), jnp.float32),
        scratch_types=[
            pltpu.VMEM((_NCHUNK, _CHUNK), jnp.int32),
            pltpu.VMEM((_NCHUNK, _CHUNK), jnp.int32),
            pltpu.VMEM((_BPW, _D), jnp.float32),
            pltpu.VMEM((_BPW, _D), jnp.float32),
            pltpu.VMEM((_L, _L), jnp.float32),
            pltpu.VMEM((_BPW,), jnp.float32),
            pltpu.SemaphoreType.DMA,
            pltpu.SemaphoreType.DMA,
        ],
        compiler_params=pltpu.CompilerParams(
            needs_layout_passes=False, use_tc_tiling_on_sc=False),
    )
    return k(user_table, item_table,
             user_ids.astype(jnp.int32), item_ids.astype(jnp.int32))
